# baseline (speedup 1.0000x reference)
import jax
import jax.numpy as jnp
from jax.experimental import pallas as pl

H = 128
W = 128
D = 128
GRID = 0.0078125
SCALE_MULT = 0.05
N = 8192
P = 1024
C = 18

BN = 512


def _agg_kernel(p_ref, o_ref, mu_ref, op_ref, sem_ref, sc_ref, cov6_ref, out_ref):
    px = p_ref[:, 0:1]
    py = p_ref[:, 1:2]
    pz = p_ref[:, 2:3]
    o = o_ref[...]
    inv_g = 1.0 / GRID
    pix = jnp.floor((px - o[:, 0:1]) * inv_g).astype(jnp.int32)
    piy = jnp.floor((py - o[:, 1:2]) * inv_g).astype(jnp.int32)
    piz = jnp.floor((pz - o[:, 2:3]) * inv_g).astype(jnp.int32)

    mux = mu_ref[0:1, :]
    muy = mu_ref[1:2, :]
    muz = mu_ref[2:3, :]
    mix = jnp.floor((mux - o[:, 0:1]) * inv_g).astype(jnp.int32)
    miy = jnp.floor((muy - o[:, 1:2]) * inv_g).astype(jnp.int32)
    miz = jnp.floor((muz - o[:, 2:3]) * inv_g).astype(jnp.int32)
    smax = jnp.max(sc_ref[...], axis=0, keepdims=True)
    radii = jnp.ceil(smax * (SCALE_MULT * inv_g)).astype(jnp.int32)

    dx = px - mux
    dy = py - muy
    dz = pz - muz

    c0 = cov6_ref[0:1, :]
    c1 = cov6_ref[1:2, :]
    c2 = cov6_ref[2:3, :]
    c3 = cov6_ref[3:4, :]
    c4 = cov6_ref[4:5, :]
    c5 = cov6_ref[5:6, :]

    power = (-0.5 * (c0 * dx * dx + c1 * dy * dy + c2 * dz * dz)
             - c3 * dx * dy - c4 * dy * dz - c5 * dx * dz)

    within = ((jnp.abs(pix - mix) <= radii)
              & (jnp.abs(piy - miy) <= radii)
              & (jnp.abs(piz - miz) <= radii))

    w = jnp.where(within, op_ref[...] * jnp.exp(power), 0.0)
    out_ref[...] = jnp.dot(w, sem_ref[...], preferred_element_type=jnp.float32)


def kernel(pts, means3D, opacities, semantics, scales, cov3D, metas, origin_use):
    p = pts[0]
    mu_t = means3D[0].T
    op_row = opacities[0][None, :]
    sem = semantics[0]
    sc_t = scales[0].T
    cov6_t = cov3D[0].reshape(P, 9)[:, jnp.array([0, 4, 8, 1, 5, 2])].T
    o = origin_use[None, :]

    grid = (N // BN,)
    out = pl.pallas_call(
        _agg_kernel,
        grid=grid,
        in_specs=[
            pl.BlockSpec((BN, 3), lambda i: (i, 0)),
            pl.BlockSpec((1, 3), lambda i: (0, 0)),
            pl.BlockSpec((3, P), lambda i: (0, 0)),
            pl.BlockSpec((1, P), lambda i: (0, 0)),
            pl.BlockSpec((P, C), lambda i: (0, 0)),
            pl.BlockSpec((3, P), lambda i: (0, 0)),
            pl.BlockSpec((6, P), lambda i: (0, 0)),
        ],
        out_specs=pl.BlockSpec((BN, C), lambda i: (i, 0)),
        out_shape=jax.ShapeDtypeStruct((N, C), jnp.float32),
    )(p, o, mu_t, op_row, sem, sc_t, cov6_t)
    return out
